# baseline (speedup 1.0000x reference)
import jax
import jax.numpy as jnp
from jax import lax
from jax.experimental import pallas as pl
from jax.experimental.pallas import tpu as pltpu

_B = 8192
_D_IN = 64
_H = 512
_D_OUT = 256
_E = 16
_TOPK = 8

_T = 4096
_GRID = _B // _T


def _moe_body(x_ref, xt_ref, wgt_t_ref, bg_ref, w1_ref, b1_ref, w2_ref,
              b2_ref, wm_ref, bm_ref, out_ref, ss_ref):
    i = pl.program_id(0)
    x = x_ref[...]

    lt = jnp.dot(wgt_t_ref[...], xt_ref[...],
                 preferred_element_type=jnp.float32)
    lt = lt + bg_ref[...]
    iota = lax.broadcasted_iota(jnp.int32, (_E, _T), 0)
    m = lt
    sel = jnp.zeros((_E, _T), jnp.bool_)
    gmax = None
    for k in range(_TOPK):
        mx = jnp.max(m, axis=0, keepdims=True)
        if k == 0:
            gmax = mx
        is_mx = m == mx
        first = jnp.min(jnp.where(is_mx, iota, _E), axis=0, keepdims=True)
        pick = iota == first
        sel = jnp.logical_or(sel, pick)
        m = jnp.where(pick, -jnp.inf, m)
    ex = jnp.where(sel, jnp.exp(lt - gmax), 0.0)
    wt = ex / jnp.sum(ex, axis=0, keepdims=True)
    wgt = wt.T

    acc = jnp.zeros((_T, _D_OUT), jnp.float32)
    for e in range(_E):
        h = jnp.tanh(
            jnp.dot(x, w1_ref[e], preferred_element_type=jnp.float32)
            + b1_ref[e:e + 1, :])
        o = (jnp.dot(h, w2_ref[e], preferred_element_type=jnp.float32)
             + b2_ref[e:e + 1, :])
        acc = acc + wgt[:, e:e + 1] * o
    c = jnp.tanh(acc)
    out_ref[...] = c

    pred = jnp.dot(c, wm_ref[...], preferred_element_type=jnp.float32)
    pred = pred + bm_ref[...]
    part = jnp.sum((c - pred) ** 2, axis=(0, 1), keepdims=True)

    @pl.when(i == 0)
    def _init():
        ss_ref[...] = jnp.zeros_like(ss_ref)

    ss_ref[...] = ss_ref[...] + part

    @pl.when(i == _GRID - 1)
    def _finish():
        ss_ref[...] = ss_ref[...] * (1.0 / (_B * _D_OUT))


def kernel(input_signal, Wg, bg, W1, b1, W2, b2, Wm, bm):
    xt = input_signal.T
    wg_t = Wg.T
    bg2 = bg.reshape(_E, 1)
    bm2 = bm.reshape(1, _D_OUT)
    full = lambda shape: pl.BlockSpec(shape, lambda i: (0,) * len(shape))
    combined, ss = pl.pallas_call(
        _moe_body,
        grid=(_GRID,),
        in_specs=[
            pl.BlockSpec((_T, _D_IN), lambda i: (i, 0)),
            pl.BlockSpec((_D_IN, _T), lambda i: (0, i)),
            full((_E, _D_IN)),
            full((_E, 1)),
            full((_E, _D_IN, _H)),
            full((_E, _H)),
            full((_E, _H, _D_OUT)),
            full((_E, _D_OUT)),
            full((_D_OUT, _D_OUT)),
            full((1, _D_OUT)),
        ],
        out_specs=[
            pl.BlockSpec((_T, _D_OUT), lambda i: (i, 0)),
            pl.BlockSpec((1, 1), lambda i: (0, 0)),
        ],
        out_shape=[
            jax.ShapeDtypeStruct((_B, _D_OUT), jnp.float32),
            jax.ShapeDtypeStruct((1, 1), jnp.float32),
        ],
        compiler_params=pltpu.CompilerParams(
            dimension_semantics=("arbitrary",)),
    )(input_signal, xt, wg_t, bg2, W1, b1, W2, b2, Wm, bm2)
    return combined, ss[0, 0]


# speedup vs baseline: 1.4591x; 1.4591x over previous
import jax
import jax.numpy as jnp
from jax import lax
from jax.experimental import pallas as pl
from jax.experimental.pallas import tpu as pltpu

_B = 8192
_D_IN = 64
_H = 512
_D_OUT = 256
_E = 16
_TOPK = 8

_T = 2048
_GRID = _B // _T


def _moe_body(x_ref, xt_ref, wgt_t_ref, bg_ref, w1_ref, b1_ref, w2_ref,
              b2_ref, wm_ref, bm_ref, out_ref, ss_ref):
    i = pl.program_id(0)
    x = x_ref[...]

    lt = jnp.dot(wgt_t_ref[...], xt_ref[...],
                 preferred_element_type=jnp.float32)
    lt = lt + bg_ref[...]
    iota = lax.broadcasted_iota(jnp.int32, (_E, _T), 0)
    m = lt
    sel = jnp.zeros((_E, _T), jnp.bool_)
    gmax = None
    for k in range(_TOPK):
        mx = jnp.max(m, axis=0, keepdims=True)
        if k == 0:
            gmax = mx
        is_mx = m == mx
        first = jnp.min(jnp.where(is_mx, iota, _E), axis=0, keepdims=True)
        pick = iota == first
        sel = jnp.logical_or(sel, pick)
        m = jnp.where(pick, -jnp.inf, m)
    ex = jnp.where(sel, jnp.exp(lt - gmax), 0.0)
    wt = ex / jnp.sum(ex, axis=0, keepdims=True)
    wgt = wt.T

    acc = jnp.zeros((_T, _D_OUT), jnp.float32)
    for e in range(_E):
        h = jnp.tanh(
            jnp.dot(x, w1_ref[e], preferred_element_type=jnp.float32)
            + b1_ref[e:e + 1, :])
        o = (jnp.dot(h, w2_ref[e], preferred_element_type=jnp.float32)
             + b2_ref[e:e + 1, :])
        acc = acc + wgt[:, e:e + 1] * o
    c = jnp.tanh(acc)
    out_ref[...] = c

    pred = jnp.dot(c, wm_ref[...], preferred_element_type=jnp.float32)
    pred = pred + bm_ref[...]
    part = jnp.sum((c - pred) ** 2, axis=(0, 1), keepdims=True)

    @pl.when(i == 0)
    def _init():
        ss_ref[...] = jnp.zeros_like(ss_ref)

    ss_ref[...] = ss_ref[...] + part

    @pl.when(i == _GRID - 1)
    def _finish():
        ss_ref[...] = ss_ref[...] * (1.0 / (_B * _D_OUT))


def kernel(input_signal, Wg, bg, W1, b1, W2, b2, Wm, bm):
    xt = input_signal.T
    wg_t = Wg.T
    bg2 = bg.reshape(_E, 1)
    bm2 = bm.reshape(1, _D_OUT)
    full = lambda shape: pl.BlockSpec(shape, lambda i: (0,) * len(shape))
    combined, ss = pl.pallas_call(
        _moe_body,
        grid=(_GRID,),
        in_specs=[
            pl.BlockSpec((_T, _D_IN), lambda i: (i, 0)),
            pl.BlockSpec((_D_IN, _T), lambda i: (0, i)),
            full((_E, _D_IN)),
            full((_E, 1)),
            full((_E, _D_IN, _H)),
            full((_E, _H)),
            full((_E, _H, _D_OUT)),
            full((_E, _D_OUT)),
            full((_D_OUT, _D_OUT)),
            full((1, _D_OUT)),
        ],
        out_specs=[
            pl.BlockSpec((_T, _D_OUT), lambda i: (i, 0)),
            pl.BlockSpec((1, 1), lambda i: (0, 0)),
        ],
        out_shape=[
            jax.ShapeDtypeStruct((_B, _D_OUT), jnp.float32),
            jax.ShapeDtypeStruct((1, 1), jnp.float32),
        ],
        compiler_params=pltpu.CompilerParams(
            dimension_semantics=("arbitrary",)),
    )(input_signal, xt, wg_t, bg2, W1, b1, W2, b2, Wm, bm2)
    return combined, ss[0, 0]


# speedup vs baseline: 1.4630x; 1.0027x over previous
import jax
import jax.numpy as jnp
from jax import lax
from jax.experimental import pallas as pl
from jax.experimental.pallas import tpu as pltpu

_B = 8192
_D_IN = 64
_H = 512
_D_OUT = 256
_E = 16
_TOPK = 8

_T = 2048
_GRID = _B // _T


def _moe_body(x_ref, wgt_t_ref, bg_ref, w1_ref, b1_ref, w2_ref,
              b2_ref, wm_ref, bm_ref, out_ref, ss_ref):
    i = pl.program_id(0)
    x = x_ref[...]

    lt = lax.dot_general(wgt_t_ref[...], x, (((1,), (1,)), ((), ())),
                         preferred_element_type=jnp.float32)
    lt = lt + bg_ref[...]
    iota = lax.broadcasted_iota(jnp.int32, (_E, _T), 0)
    m = lt
    sel = jnp.zeros((_E, _T), jnp.bool_)
    gmax = None
    for k in range(_TOPK):
        mx = jnp.max(m, axis=0, keepdims=True)
        if k == 0:
            gmax = mx
        is_mx = m == mx
        first = jnp.min(jnp.where(is_mx, iota, _E), axis=0, keepdims=True)
        pick = iota == first
        sel = jnp.logical_or(sel, pick)
        m = jnp.where(pick, -jnp.inf, m)
    ex = jnp.where(sel, jnp.exp(lt - gmax), 0.0)
    wt = ex / jnp.sum(ex, axis=0, keepdims=True)
    wgt = wt.T

    acc = jnp.zeros((_T, _D_OUT), jnp.float32)
    for e in range(_E):
        h = jnp.tanh(
            jnp.dot(x, w1_ref[e], preferred_element_type=jnp.float32)
            + b1_ref[e:e + 1, :])
        o = (jnp.dot(h, w2_ref[e], preferred_element_type=jnp.float32)
             + b2_ref[e:e + 1, :])
        acc = acc + wgt[:, e:e + 1] * o
    c = jnp.tanh(acc)
    out_ref[...] = c

    pred = jnp.dot(c, wm_ref[...], preferred_element_type=jnp.float32)
    pred = pred + bm_ref[...]
    part = jnp.sum((c - pred) ** 2, axis=(0, 1), keepdims=True)

    @pl.when(i == 0)
    def _init():
        ss_ref[...] = jnp.zeros_like(ss_ref)

    ss_ref[...] = ss_ref[...] + part

    @pl.when(i == _GRID - 1)
    def _finish():
        ss_ref[...] = ss_ref[...] * (1.0 / (_B * _D_OUT))


def kernel(input_signal, Wg, bg, W1, b1, W2, b2, Wm, bm):
    wg_t = Wg.T
    bg2 = bg.reshape(_E, 1)
    bm2 = bm.reshape(1, _D_OUT)
    full = lambda shape: pl.BlockSpec(shape, lambda i: (0,) * len(shape))
    combined, ss = pl.pallas_call(
        _moe_body,
        grid=(_GRID,),
        in_specs=[
            pl.BlockSpec((_T, _D_IN), lambda i: (i, 0)),
            full((_E, _D_IN)),
            full((_E, 1)),
            full((_E, _D_IN, _H)),
            full((_E, _H)),
            full((_E, _H, _D_OUT)),
            full((_E, _D_OUT)),
            full((_D_OUT, _D_OUT)),
            full((1, _D_OUT)),
        ],
        out_specs=[
            pl.BlockSpec((_T, _D_OUT), lambda i: (i, 0)),
            pl.BlockSpec((1, 1), lambda i: (0, 0)),
        ],
        out_shape=[
            jax.ShapeDtypeStruct((_B, _D_OUT), jnp.float32),
            jax.ShapeDtypeStruct((1, 1), jnp.float32),
        ],
        compiler_params=pltpu.CompilerParams(
            dimension_semantics=("arbitrary",)),
    )(input_signal, wg_t, bg2, W1, b1, W2, b2, Wm, bm2)
    return combined, ss[0, 0]
